# baseline (speedup 1.0000x reference)
import jax
import jax.numpy as jnp
from jax.experimental import pallas as pl
from jax.experimental.pallas import tpu as pltpu

_BB = 16


def _body(rows_sref, cols_sref, tok_ref, re_ref, ce_ref, out_ref, pos_ref):
    i = pl.program_id(0)

    @pl.when(i == 0)
    def _():
        def loop_body(s, _):
            pos_ref[s, :] = re_ref[rows_sref[s], :] + ce_ref[cols_sref[s], :]
            return ()

        jax.lax.fori_loop(0, pos_ref.shape[0], loop_body, ())

    out_ref[...] = tok_ref[...] + pos_ref[...][None, :, :]


def kernel(tokens, row_embed, col_embed, rows, cols):
    B, S, D = tokens.shape
    grid_spec = pltpu.PrefetchScalarGridSpec(
        num_scalar_prefetch=2,
        grid=(B // _BB,),
        in_specs=[
            pl.BlockSpec((_BB, S, D), lambda i, r, c: (i, 0, 0)),
            pl.BlockSpec((8, D), lambda i, r, c: (0, 0)),
            pl.BlockSpec((8, D), lambda i, r, c: (0, 0)),
        ],
        out_specs=pl.BlockSpec((_BB, S, D), lambda i, r, c: (i, 0, 0)),
        scratch_shapes=[pltpu.VMEM((S, D), jnp.float32)],
    )
    return pl.pallas_call(
        _body,
        grid_spec=grid_spec,
        out_shape=jax.ShapeDtypeStruct((B, S, D), tokens.dtype),
        compiler_params=pltpu.CompilerParams(
            dimension_semantics=("arbitrary",),
        ),
    )(rows.astype(jnp.int32), cols.astype(jnp.int32),
      tokens, row_embed, col_embed)


# speedup vs baseline: 1.0084x; 1.0084x over previous
import jax
import jax.numpy as jnp
from jax.experimental import pallas as pl
from jax.experimental.pallas import tpu as pltpu

_BB = 32


def _body(rows_sref, cols_sref, tok_ref, re_ref, ce_ref, out_ref, pos_ref):
    i = pl.program_id(0)

    @pl.when(i == 0)
    def _():
        def loop_body(s, _):
            pos_ref[s, :] = re_ref[rows_sref[s], :] + ce_ref[cols_sref[s], :]
            return ()

        jax.lax.fori_loop(0, pos_ref.shape[0], loop_body, ())

    out_ref[...] = tok_ref[...] + pos_ref[...][None, :, :]


def kernel(tokens, row_embed, col_embed, rows, cols):
    B, S, D = tokens.shape
    grid_spec = pltpu.PrefetchScalarGridSpec(
        num_scalar_prefetch=2,
        grid=(B // _BB,),
        in_specs=[
            pl.BlockSpec((_BB, S, D), lambda i, r, c: (i, 0, 0)),
            pl.BlockSpec((8, D), lambda i, r, c: (0, 0)),
            pl.BlockSpec((8, D), lambda i, r, c: (0, 0)),
        ],
        out_specs=pl.BlockSpec((_BB, S, D), lambda i, r, c: (i, 0, 0)),
        scratch_shapes=[pltpu.VMEM((S, D), jnp.float32)],
    )
    return pl.pallas_call(
        _body,
        grid_spec=grid_spec,
        out_shape=jax.ShapeDtypeStruct((B, S, D), tokens.dtype),
        compiler_params=pltpu.CompilerParams(
            dimension_semantics=("arbitrary",),
        ),
    )(rows.astype(jnp.int32), cols.astype(jnp.int32),
      tokens, row_embed, col_embed)
